# baseline (device time: 173006 ns/iter reference)
import functools

import jax
import jax.numpy as jnp
from jax import lax
from jax.experimental import pallas as pl
from jax.experimental.pallas import tpu as pltpu

W = 8
B_LOC = 2
SQ = 512
SKV = 512
H_LOC = 8
DH = 64
D = 768


def kernel(x, Wq, K_ext, V_ext, Wo):
    my = lax.axis_index("i")

    def prep(ext):
        loc = lax.dynamic_slice_in_dim(ext, my * B_LOC, B_LOC, axis=0)
        return loc.astype(jnp.bfloat16).transpose(0, 2, 3, 1)

    k_loc = prep(K_ext)
    v_loc = prep(V_ext)
    x_bf = x.astype(jnp.bfloat16).reshape(B_LOC * SQ, D)
    wq_bf = (Wq * (0.125 * 1.4426950408889634)).astype(jnp.bfloat16)
    wo_bf = Wo.astype(jnp.bfloat16)

    def body(x_ref, wq_ref, k_ref, v_ref, wo_ref, out_ref,
             wq_hops, wo_hops, kv_k, kv_v, bias_ref, ctx_buf,
             kv_sems, wq_ssem, wq_rsem, wo_ssem, wo_rsem):
        me = lax.axis_index("i")
        right = lax.rem(me + 1, W)
        left = lax.rem(me + W - 1, W)

        wq_hops[0] = wq_ref[...]
        wo_hops[0] = wo_ref[...]

        kv_k[...] = jnp.zeros(kv_k.shape, jnp.bfloat16)
        kv_v[...] = jnp.zeros(kv_v.shape, jnp.bfloat16)

        def issue_kv(hop, slot):
            jj = lax.rem(me - hop + W, W)
            waits = []
            for b in range(B_LOC):
                for p in range(H_LOC // 2):
                    for i in range(2):
                        hg = jj * H_LOC + 2 * p + i
                        blk = (slice(i * DH, (i + 1) * DH),
                               slice(i * SKV, (i + 1) * SKV))
                        for src, buf in ((k_ref, kv_k), (v_ref, kv_v)):
                            c = pltpu.make_async_copy(
                                src.at[b, hg],
                                buf.at[(slot, b, p) + blk],
                                kv_sems.at[slot])
                            c.start()
                            waits.append(c)
            return waits

        kv_waits = issue_kv(0, 0)

        qi = lax.broadcasted_iota(jnp.int32, (SQ, SKV), 0)
        ki = lax.broadcasted_iota(jnp.int32, (SQ, SKV), 1)
        mask = (jnp.abs(qi - ki) <= 128) | (ki < 32) | (qi < 32)
        m1 = jnp.where(mask, 1.0, 0.0).astype(jnp.bfloat16)
        bias_ref[:, :SKV] = m1
        bias_ref[:, SKV:] = m1

        bar = pltpu.get_barrier_semaphore()
        for nbr in (left, right):
            pl.semaphore_signal(bar, inc=1, device_id=(nbr,),
                                device_id_type=pl.DeviceIdType.MESH)
        pl.semaphore_wait(bar, 2)

        prev_rdma = []
        for hop in range(W):
            slot = hop % 2
            for r in prev_rdma:
                r.wait()

            if hop < W - 1:
                r1 = pltpu.make_async_remote_copy(
                    src_ref=wq_hops.at[hop], dst_ref=wq_hops.at[hop + 1],
                    send_sem=wq_ssem.at[hop], recv_sem=wq_rsem.at[hop],
                    device_id=(right,), device_id_type=pl.DeviceIdType.MESH)
                r2 = pltpu.make_async_remote_copy(
                    src_ref=wo_hops.at[hop], dst_ref=wo_hops.at[hop + 1],
                    send_sem=wo_ssem.at[hop], recv_sem=wo_rsem.at[hop],
                    device_id=(right,), device_id_type=pl.DeviceIdType.MESH)
                r1.start()
                r2.start()
                prev_rdma = [r1, r2]

            for c in kv_waits:
                c.wait()
            if hop < W - 1:
                kv_waits = issue_kv(hop + 1, 1 - slot)

            wq_j = wq_hops[hop]
            wo_j = wo_hops[hop]
            q_all = jnp.dot(x_ref[...], wq_j,
                            preferred_element_type=jnp.float32
                            ).astype(jnp.bfloat16)
            for b in range(B_LOC):
                for p in range(H_LOC // 2):
                    q2 = q_all[b * SQ:(b + 1) * SQ,
                               p * 2 * DH:(p + 1) * 2 * DH]
                    kbd = kv_k[slot, b, p]
                    vbd = kv_v[slot, b, p]
                    s2 = jnp.dot(q2, kbd,
                                 preferred_element_type=jnp.float32)
                    wb = jnp.exp2(s2).astype(jnp.bfloat16) * bias_ref[...]
                    d0 = jnp.sum(wb[:, :SKV], axis=1, keepdims=True,
                                 dtype=jnp.float32)
                    d1 = jnp.sum(wb[:, SKV:], axis=1, keepdims=True,
                                 dtype=jnp.float32)
                    ctx2 = lax.dot_general(
                        wb, vbd, (((1,), (1,)), ((), ())),
                        preferred_element_type=jnp.float32)
                    col = b * SQ, p * 2 * DH
                    ctx_buf[col[0]:col[0] + SQ,
                            col[1]:col[1] + DH] = (
                        ctx2[:, :DH] / d0).astype(jnp.bfloat16)
                    ctx_buf[col[0]:col[0] + SQ,
                            col[1] + DH:col[1] + 2 * DH] = (
                        ctx2[:, DH:] / d1).astype(jnp.bfloat16)
            acc = jnp.dot(ctx_buf[...], wo_j,
                          preferred_element_type=jnp.float32)
            if hop == 0:
                out_ref[...] = acc
            else:
                out_ref[...] = out_ref[...] + acc

        @functools.partial(pl.run_scoped,
                           exit_bar=pltpu.SemaphoreType.REGULAR)
        def _(exit_bar):
            for nbr in (left, right):
                pl.semaphore_signal(exit_bar, inc=1, device_id=(nbr,),
                                    device_id_type=pl.DeviceIdType.MESH)
            pl.semaphore_wait(exit_bar, 2)

    out = pl.pallas_call(
        body,
        out_shape=jax.ShapeDtypeStruct((B_LOC * SQ, D), jnp.float32),
        in_specs=[
            pl.BlockSpec(memory_space=pltpu.VMEM),
            pl.BlockSpec(memory_space=pltpu.VMEM),
            pl.BlockSpec(memory_space=pltpu.MemorySpace.HBM),
            pl.BlockSpec(memory_space=pltpu.MemorySpace.HBM),
            pl.BlockSpec(memory_space=pltpu.VMEM),
        ],
        out_specs=pl.BlockSpec(memory_space=pltpu.VMEM),
        scratch_shapes=[
            pltpu.VMEM((W, D, H_LOC * DH), jnp.bfloat16),
            pltpu.VMEM((W, H_LOC * DH, D), jnp.bfloat16),
            pltpu.VMEM((2, B_LOC, H_LOC // 2, 2 * DH, 2 * SKV),
                       jnp.bfloat16),
            pltpu.VMEM((2, B_LOC, H_LOC // 2, 2 * DH, 2 * SKV),
                       jnp.bfloat16),
            pltpu.VMEM((SQ, 2 * SKV), jnp.bfloat16),
            pltpu.VMEM((B_LOC * SQ, H_LOC * DH), jnp.bfloat16),
            pltpu.SemaphoreType.DMA((2,)),
            pltpu.SemaphoreType.DMA((W - 1,)),
            pltpu.SemaphoreType.DMA((W - 1,)),
            pltpu.SemaphoreType.DMA((W - 1,)),
            pltpu.SemaphoreType.DMA((W - 1,)),
        ],
        compiler_params=pltpu.CompilerParams(
            collective_id=0, vmem_limit_bytes=60 * 1024 * 1024),
    )(x_bf, wq_bf, k_loc, v_loc, wo_bf)
    return out.reshape(B_LOC, SQ, D)
